# baseline (speedup 1.0000x reference)
import jax
import jax.numpy as jnp
from jax import lax
from jax.experimental import pallas as pl
from jax.experimental.pallas import tpu as pltpu
from jax.experimental.pallas import tpu_sc as plsc

BS = 8
NC = 10
T = 32
F = 2048
H = 2048
MEM = 60
NVID = 2 * BS
RPV = NC * T
ROWS = NVID * RPV
K = T // 16 + 1
HF = F // 2
NEG = -3.0e38


def _mlp_kernel(x_ref, wa1_ref, ba1_ref, wa2_ref, ba2_ref, w1_ref, b1_ref,
                w2_ref, b2_ref, w3_ref, b3_ref, out_ref, score_ref, fmag_ref):
    x = x_ref[...]
    h = jnp.maximum(
        jax.lax.dot(x, wa1_ref[...], preferred_element_type=jnp.float32)
        + ba1_ref[...], 0.0)
    out = x + jax.lax.dot(h, wa2_ref[...],
                          preferred_element_type=jnp.float32) + ba2_ref[...]
    out_ref[...] = out
    fmag_ref[...] = jnp.sqrt(jnp.sum(out * out, axis=1, keepdims=True))
    s = jnp.maximum(
        jax.lax.dot(out, w1_ref[...], preferred_element_type=jnp.float32)
        + b1_ref[...], 0.0)
    s = jnp.maximum(
        jax.lax.dot(s, w2_ref[...], preferred_element_type=jnp.float32)
        + b2_ref[...], 0.0)
    score_ref[...] = jax.nn.sigmoid(
        jax.lax.dot(s, w3_ref[...], preferred_element_type=jnp.float32)
        + b3_ref[...])


QF = F // 4
VH = NVID // 2


def _make_sc_select(v0):
    def _sc_select(feat_ref, fmag_ref, score_ref, mask_ref,
                   sel_out, scm_out, wsel_out,
                   fmag_v, mask_v, score_v, vec32_v, idx_v, rows_v, sel_v,
                   sem):
        wid = lax.axis_index("s") * 2 + lax.axis_index("c")
        rel = wid // 4
        q = wid - 4 * rel

        pltpu.sync_copy(fmag_ref.at[pl.ds(rel * RPV, RPV)], fmag_v)
        pltpu.sync_copy(mask_ref.at[pl.ds(rel * RPV, RPV)], mask_v)
        pltpu.sync_copy(score_ref.at[pl.ds(rel * RPV, RPV)], score_v)

        tlo = lax.iota(jnp.int32, 16)
        zero = jnp.zeros((16,), jnp.float32)
        one = jnp.full((16,), 1.0)

        def crop_mean(ref, off):
            acc = zero
            for c in range(NC):
                acc = acc + ref[pl.ds(c * T + off, 16)]
            return acc * (1.0 / NC)

        fm0 = crop_mean(fmag_v, 0)
        fm1 = crop_mean(fmag_v, 16)
        mk0 = crop_mean(mask_v, 0)
        mk1 = crop_mean(mask_v, 16)
        a = fm0 * mk0
        b = fm1 * mk1

        red_v = vec32_v

        def rot(x, k):
            red_v[pl.ds(0, 16)] = x
            red_v[pl.ds(16, 16)] = x
            return red_v[pl.ds(k, 16)]

        def max_splat(x):
            for k in (8, 4, 2, 1):
                x = jnp.maximum(x, rot(x, k))
            return x

        def min_splat(x):
            for k in (8, 4, 2, 1):
                x = jnp.minimum(x, rot(x, k))
            return x

        tlo_f = tlo.astype(jnp.float32)
        ts = []
        for _ in range(K):
            ma = max_splat(a)
            mb = max_splat(b)
            pa_f = jnp.where(ma >= mb, one, zero)
            ca = min_splat(jnp.where(a >= ma, tlo_f, 99.0))
            cb = min_splat(jnp.where(b >= mb, tlo_f, 99.0))
            t_f = pa_f * ca + (one - pa_f) * (cb + 16.0)
            onehot_a = jnp.where(tlo_f == ca, one, zero) * pa_f
            onehot_b = jnp.where(tlo_f == cb, one, zero) * (one - pa_f)
            a = a + onehot_a * NEG
            b = b + onehot_b * NEG
            ts.append(t_f)

        @pl.when(q == 0)
        def _():
            sm0 = crop_mean(score_v, 0)
            sm1 = crop_mean(score_v, 16)
            vec32_v[pl.ds(0, 16)] = sm0
            vec32_v[pl.ds(16, 16)] = sm1
            pltpu.sync_copy(vec32_v, scm_out.at[rel])

        @pl.when(q == 1)
        def _():
            w0 = zero
            w1 = zero
            for t in ts:
                w0 = w0 + jnp.where(tlo_f == t, 1.0, 0.0)
                w1 = w1 + jnp.where(tlo_f + 16.0 == t, 1.0, 0.0)
            vec32_v[pl.ds(0, 16)] = w0
            vec32_v[pl.ds(16, 16)] = w1
            pltpu.sync_copy(vec32_v, wsel_out.at[rel])

        ts_i = [t.astype(jnp.int32) for t in ts]
        for g in range(2):
            j = tlo + 16 * g
            c = lax.shift_right_logical(j * 11, 5)
            kk = j - c * 3
            t = jnp.where(kk == 0, ts_i[0],
                          jnp.where(kk == 1, ts_i[1], ts_i[2]))
            row = 4 * (rel * RPV + c * T + t) + q
            row = jnp.where(j < NC * K, row, q)
            idx_v[pl.ds(16 * g, 16)] = row
        pltpu.async_copy(feat_ref.at[idx_v], rows_v, sem).wait()

        def body(s, carry):
            off = s * 16
            for c in range(NC):
                acc = (rows_v[3 * c, pl.ds(off, 16)]
                       + rows_v[3 * c + 1, pl.ds(off, 16)]
                       + rows_v[3 * c + 2, pl.ds(off, 16)])
                sel_v[c, pl.ds(off, 16)] = acc * (1.0 / K)
            return carry

        lax.fori_loop(0, QF // 16, body, 0)

        for c in range(NC):
            pltpu.sync_copy(sel_v.at[c], sel_out.at[4 * (rel * NC + c) + q])

    return _sc_select


def _sc_select_half(out_half, fmag_half, score_half, mask_half, v0):
    fn = pl.kernel(
        _make_sc_select(v0),
        mesh=plsc.VectorSubcoreMesh(core_axis_name="c", subcore_axis_name="s"),
        out_type=[
            jax.ShapeDtypeStruct((4 * VH * NC, QF), jnp.float32),
            jax.ShapeDtypeStruct((VH, T), jnp.float32),
            jax.ShapeDtypeStruct((VH, T), jnp.float32),
        ],
        scratch_types=[
            pltpu.VMEM((RPV,), jnp.float32),
            pltpu.VMEM((RPV,), jnp.float32),
            pltpu.VMEM((RPV,), jnp.float32),
            pltpu.VMEM((T,), jnp.float32),
            pltpu.VMEM((2 * 16,), jnp.int32),
            pltpu.VMEM((2 * 16, QF), jnp.float32),
            pltpu.VMEM((NC, QF), jnp.float32),
            pltpu.SemaphoreType.DMA,
        ],
    )
    sel4, scm, wsel = fn(out_half.reshape(4 * VH * RPV, QF),
                         fmag_half, score_half, mask_half)
    return sel4.reshape(VH * NC, F), scm, wsel


def _loss_kernel(sel_ref, scm_ref, wsel_ref, mem_ref,
                 ssel_ref, fs_ref, sacc_ref, cacc_ref, u_ref):
    sel = sel_ref[...]
    fs_ref[...] = jnp.sqrt(jnp.sum(sel * sel, axis=1, keepdims=True))
    scm = scm_ref[...]
    w = wsel_ref[...]
    ssel_ref[...] = jnp.sum(scm * w, axis=1, keepdims=True) / K

    mem = mem_ref[...]
    logits = jax.lax.dot_general(sel, mem, (((1,), (1,)), ((), ())),
                                 preferred_element_type=jnp.float32)
    att = jax.nn.softmax(logits, axis=-1)
    recon = jax.lax.dot(att, mem, preferred_element_type=jnp.float32)
    diff = recon - sel
    sacc_ref[...] = (jnp.sum(diff * diff) / (NVID * NC * F)).reshape(1, 1)
    cacc_ref[...] = (-jnp.sum(att * jnp.log(att + 1e-8))
                     / (NVID * NC)).reshape(1, 1)

    mn = mem / (jnp.sqrt(jnp.sum(mem * mem, axis=1, keepdims=True)) + 1e-8)
    g = jax.lax.dot_general(mn, mn, (((1,), (1,)), ((), ())),
                            preferred_element_type=jnp.float32)
    r = jax.lax.broadcasted_iota(jnp.int32, (MEM, MEM), 0)
    c = jax.lax.broadcasted_iota(jnp.int32, (MEM, MEM), 1)
    gd = g - jnp.where(r == c, 1.0, 0.0)
    u_ref[...] = (jnp.sum(gd * gd) / (MEM * MEM)).reshape(1, 1)


@jax.jit
def kernel(inputs, mask, Wa1, ba1, Wa2, ba2, W1, b1, W2, b2, W3, b3, Mem):
    x = inputs.reshape(ROWS, F)
    R = 512
    HR = ROWS // 2

    def run_mlp(x_half):
        return pl.pallas_call(
            _mlp_kernel,
            grid=(HR // R,),
            in_specs=[
                pl.BlockSpec((R, F), lambda i: (i, 0)),
                pl.BlockSpec((F, H), lambda i: (0, 0)),
                pl.BlockSpec((1, H), lambda i: (0, 0)),
                pl.BlockSpec((H, F), lambda i: (0, 0)),
                pl.BlockSpec((1, F), lambda i: (0, 0)),
                pl.BlockSpec((F, 512), lambda i: (0, 0)),
                pl.BlockSpec((1, 512), lambda i: (0, 0)),
                pl.BlockSpec((512, 128), lambda i: (0, 0)),
                pl.BlockSpec((1, 128), lambda i: (0, 0)),
                pl.BlockSpec((128, 1), lambda i: (0, 0)),
                pl.BlockSpec((1, 1), lambda i: (0, 0)),
            ],
            out_specs=[
                pl.BlockSpec((R, F), lambda i: (i, 0)),
                pl.BlockSpec((R, 1), lambda i: (i, 0)),
                pl.BlockSpec((R, 1), lambda i: (i, 0)),
            ],
            out_shape=[
                jax.ShapeDtypeStruct((HR, F), jnp.float32),
                jax.ShapeDtypeStruct((HR, 1), jnp.float32),
                jax.ShapeDtypeStruct((HR, 1), jnp.float32),
            ],
        )(x_half, Wa1, ba1.reshape(1, H), Wa2, ba2.reshape(1, F), W1,
          b1.reshape(1, 512), W2, b2.reshape(1, 128), W3, b3.reshape(1, 1))

    mask_flat = mask.reshape(ROWS)
    out1, score1, fmag1 = run_mlp(x[:HR])
    sel1, scm1, wsel1 = _sc_select_half(
        out1, fmag1.reshape(HR), score1.reshape(HR), mask_flat[:HR], 0)
    out2, score2, fmag2 = run_mlp(x[HR:])
    sel2, scm2, wsel2 = _sc_select_half(
        out2, fmag2.reshape(HR), score2.reshape(HR), mask_flat[HR:], VH)

    sel = jnp.concatenate([sel1, sel2], axis=0)
    scm = jnp.concatenate([scm1, scm2], axis=0)
    wsel = jnp.concatenate([wsel1, wsel2], axis=0)

    ssel, fs, s_loss, c_loss, u_loss = pl.pallas_call(
        _loss_kernel,
        grid=(1,),
        in_specs=[
            pl.BlockSpec((NVID * NC, F), lambda i: (0, 0)),
            pl.BlockSpec((NVID, T), lambda i: (0, 0)),
            pl.BlockSpec((NVID, T), lambda i: (0, 0)),
            pl.BlockSpec((MEM, F), lambda i: (0, 0)),
        ],
        out_specs=[
            pl.BlockSpec((NVID, 1), lambda i: (0, 0)),
            pl.BlockSpec((NVID * NC, 1), lambda i: (0, 0)),
            pl.BlockSpec((1, 1), lambda i: (0, 0)),
            pl.BlockSpec((1, 1), lambda i: (0, 0)),
            pl.BlockSpec((1, 1), lambda i: (0, 0)),
        ],
        out_shape=[
            jax.ShapeDtypeStruct((NVID, 1), jnp.float32),
            jax.ShapeDtypeStruct((NVID * NC, 1), jnp.float32),
            jax.ShapeDtypeStruct((1, 1), jnp.float32),
            jax.ShapeDtypeStruct((1, 1), jnp.float32),
            jax.ShapeDtypeStruct((1, 1), jnp.float32),
        ],
    )(sel, scm, wsel, Mem)

    scores = scm.reshape(NVID, T, 1)
    score_normal = ssel[:BS]
    score_abnormal = ssel[BS:]
    fs_flat = fs.reshape(NVID, NC)
    feat_select_normal = fs_flat[:BS].reshape(BS * NC)
    feat_select_abn = fs_flat[BS:].reshape(BS * NC)
    return (score_abnormal, score_normal, feat_select_abn, feat_select_normal,
            scores, s_loss.reshape(()), c_loss.reshape(()), u_loss.reshape(()))


# speedup vs baseline: 1.1795x; 1.1795x over previous
import jax
import jax.numpy as jnp
from jax import lax
from jax.experimental import pallas as pl
from jax.experimental.pallas import tpu as pltpu
from jax.experimental.pallas import tpu_sc as plsc

BS = 8
NC = 10
T = 32
F = 2048
H = 2048
MEM = 60
NVID = 2 * BS
RPV = NC * T
ROWS = NVID * RPV
K = T // 16 + 1
HF = F // 2
NSEL = NC * K


def _mlp_kernel(x_ref, mask_ref, wa1_ref, ba1_ref, wa2_ref, ba2_ref, w1_ref,
                b1_ref, w2_ref, b2_ref, w3_ref, b3_ref,
                out_ref, scm_ref, ssel_ref, idx_ref):
    i = pl.program_id(0)
    x = x_ref[...]
    h = jnp.maximum(
        jax.lax.dot(x, wa1_ref[...], preferred_element_type=jnp.float32)
        + ba1_ref[...], 0.0)
    out = x + jax.lax.dot(h, wa2_ref[...],
                          preferred_element_type=jnp.float32) + ba2_ref[...]
    out_ref[...] = out
    fmag = jnp.sqrt(jnp.sum(out * out, axis=1, keepdims=True))
    s = jnp.maximum(
        jax.lax.dot(out, w1_ref[...], preferred_element_type=jnp.float32)
        + b1_ref[...], 0.0)
    s = jnp.maximum(
        jax.lax.dot(s, w2_ref[...], preferred_element_type=jnp.float32)
        + b2_ref[...], 0.0)
    sc = jax.nn.sigmoid(
        jax.lax.dot(s, w3_ref[...], preferred_element_type=jnp.float32)
        + b3_ref[...])

    r_iota = jax.lax.broadcasted_iota(jnp.int32, (RPV, T), 0)
    t_iota = jax.lax.broadcasted_iota(jnp.int32, (RPV, T), 1)
    cm_mat = jnp.where(r_iota % T == t_iota, 1.0 / NC, 0.0)
    cdims = (((0,), (0,)), ((), ()))
    score_mean = jax.lax.dot_general(sc, cm_mat, cdims,
                                     precision=jax.lax.Precision.HIGHEST,
                                     preferred_element_type=jnp.float32)
    fmag_mean = jax.lax.dot_general(fmag, cm_mat, cdims,
                                    precision=jax.lax.Precision.HIGHEST,
                                    preferred_element_type=jnp.float32)
    scm_ref[0] = score_mean

    mask_mean = jnp.mean(mask_ref[0], axis=0, keepdims=True)
    drop = fmag_mean * mask_mean

    iota = jax.lax.broadcasted_iota(jnp.int32, (1, T), 1)
    v = drop
    wsel = jnp.zeros((1, T), jnp.float32)
    tks = []
    for _ in range(K):
        m = jnp.max(v, axis=1, keepdims=True)
        tk = jnp.min(jnp.where(v >= m, iota, T), axis=1, keepdims=True)
        hit = iota == tk
        wsel = wsel + jnp.where(hit, 1.0, 0.0)
        v = jnp.where(hit, -jnp.inf, v)
        tks.append(tk)

    ssel_ref[0] = jnp.sum(score_mean * wsel, axis=1, keepdims=True) / K

    col = jax.lax.broadcasted_iota(jnp.int32, (1, T), 1)
    c = jax.lax.shift_right_logical(col * 11, 5)
    kk = col - c * 3
    t = jnp.where(kk == 0, tks[0], jnp.where(kk == 1, tks[1], tks[2]))
    row = i * RPV + c * T + t
    idx_ref[0] = jnp.where(col < NSEL, row, 0)


def _sc_gather(feat_ref, idxrows_ref, sel_out, idx_vmem, idx_v, rows_v, sel_v,
               sem):
    wid = lax.axis_index("s") * 2 + lax.axis_index("c")
    v = wid // 2
    half = wid - 2 * v

    pltpu.sync_copy(idxrows_ref.at[v], idx_vmem)
    for g in range(2):
        r = idx_vmem[pl.ds(16 * g, 16)]
        idx_v[pl.ds(16 * g, 16)] = 2 * r + half
    pltpu.async_copy(feat_ref.at[idx_v], rows_v, sem).wait()

    def body(s, carry):
        off = s * 16
        for c in range(NC):
            acc = (rows_v[3 * c, pl.ds(off, 16)]
                   + rows_v[3 * c + 1, pl.ds(off, 16)]
                   + rows_v[3 * c + 2, pl.ds(off, 16)])
            sel_v[c, pl.ds(off, 16)] = acc * (1.0 / K)
        return carry

    lax.fori_loop(0, HF // 16, body, 0)

    for c in range(NC):
        pltpu.sync_copy(sel_v.at[c], sel_out.at[2 * (v * NC + c) + half])


def _loss_kernel(sel_ref, mem_ref, fs_ref, sacc_ref, cacc_ref, u_ref):
    sel = sel_ref[...]
    fs_ref[...] = jnp.sqrt(jnp.sum(sel * sel, axis=1, keepdims=True))

    mem = mem_ref[...]
    logits = jax.lax.dot_general(sel, mem, (((1,), (1,)), ((), ())),
                                 preferred_element_type=jnp.float32)
    att = jax.nn.softmax(logits, axis=-1)
    recon = jax.lax.dot(att, mem, preferred_element_type=jnp.float32)
    diff = recon - sel
    sacc_ref[...] = (jnp.sum(diff * diff) / (NVID * NC * F)).reshape(1, 1)
    cacc_ref[...] = (-jnp.sum(att * jnp.log(att + 1e-8))
                     / (NVID * NC)).reshape(1, 1)

    mn = mem / (jnp.sqrt(jnp.sum(mem * mem, axis=1, keepdims=True)) + 1e-8)
    g = jax.lax.dot_general(mn, mn, (((1,), (1,)), ((), ())),
                            preferred_element_type=jnp.float32)
    r = jax.lax.broadcasted_iota(jnp.int32, (MEM, MEM), 0)
    c = jax.lax.broadcasted_iota(jnp.int32, (MEM, MEM), 1)
    gd = g - jnp.where(r == c, 1.0, 0.0)
    u_ref[...] = (jnp.sum(gd * gd) / (MEM * MEM)).reshape(1, 1)


@jax.jit
def kernel(inputs, mask, Wa1, ba1, Wa2, ba2, W1, b1, W2, b2, W3, b3, Mem):
    x = inputs.reshape(ROWS, F)
    mask3 = mask.reshape(NVID, NC, T)

    out, scm3, ssel3, idxrows3 = pl.pallas_call(
        _mlp_kernel,
        grid=(NVID,),
        in_specs=[
            pl.BlockSpec((RPV, F), lambda i: (i, 0)),
            pl.BlockSpec((1, NC, T), lambda i: (i, 0, 0)),
            pl.BlockSpec((F, H), lambda i: (0, 0)),
            pl.BlockSpec((1, H), lambda i: (0, 0)),
            pl.BlockSpec((H, F), lambda i: (0, 0)),
            pl.BlockSpec((1, F), lambda i: (0, 0)),
            pl.BlockSpec((F, 512), lambda i: (0, 0)),
            pl.BlockSpec((1, 512), lambda i: (0, 0)),
            pl.BlockSpec((512, 128), lambda i: (0, 0)),
            pl.BlockSpec((1, 128), lambda i: (0, 0)),
            pl.BlockSpec((128, 1), lambda i: (0, 0)),
            pl.BlockSpec((1, 1), lambda i: (0, 0)),
        ],
        out_specs=[
            pl.BlockSpec((RPV, F), lambda i: (i, 0)),
            pl.BlockSpec((1, 1, T), lambda i: (i, 0, 0)),
            pl.BlockSpec((1, 1, 1), lambda i: (i, 0, 0)),
            pl.BlockSpec((1, 1, T), lambda i: (i, 0, 0)),
        ],
        out_shape=[
            jax.ShapeDtypeStruct((ROWS, F), jnp.float32),
            jax.ShapeDtypeStruct((NVID, 1, T), jnp.float32),
            jax.ShapeDtypeStruct((NVID, 1, 1), jnp.float32),
            jax.ShapeDtypeStruct((NVID, 1, T), jnp.int32),
        ],
    )(x, mask3, Wa1, ba1.reshape(1, H), Wa2, ba2.reshape(1, F), W1,
      b1.reshape(1, 512), W2, b2.reshape(1, 128), W3, b3.reshape(1, 1))

    sc_fn = pl.kernel(
        _sc_gather,
        mesh=plsc.VectorSubcoreMesh(core_axis_name="c", subcore_axis_name="s"),
        out_type=jax.ShapeDtypeStruct((2 * NVID * NC, HF), jnp.float32),
        scratch_types=[
            pltpu.VMEM((T,), jnp.int32),
            pltpu.VMEM((T,), jnp.int32),
            pltpu.VMEM((T, HF), jnp.float32),
            pltpu.VMEM((NC, HF), jnp.float32),
            pltpu.SemaphoreType.DMA,
        ],
    )
    sel2 = sc_fn(out.reshape(2 * ROWS, HF), idxrows3.reshape(NVID, T))
    sel = sel2.reshape(NVID * NC, F)

    fs, s_loss, c_loss, u_loss = pl.pallas_call(
        _loss_kernel,
        grid=(1,),
        in_specs=[
            pl.BlockSpec((NVID * NC, F), lambda i: (0, 0)),
            pl.BlockSpec((MEM, F), lambda i: (0, 0)),
        ],
        out_specs=[
            pl.BlockSpec((NVID * NC, 1), lambda i: (0, 0)),
            pl.BlockSpec((1, 1), lambda i: (0, 0)),
            pl.BlockSpec((1, 1), lambda i: (0, 0)),
            pl.BlockSpec((1, 1), lambda i: (0, 0)),
        ],
        out_shape=[
            jax.ShapeDtypeStruct((NVID * NC, 1), jnp.float32),
            jax.ShapeDtypeStruct((1, 1), jnp.float32),
            jax.ShapeDtypeStruct((1, 1), jnp.float32),
            jax.ShapeDtypeStruct((1, 1), jnp.float32),
        ],
    )(sel, Mem)

    scores = scm3.reshape(NVID, T, 1)
    ssel_flat = ssel3.reshape(NVID, 1)
    score_normal = ssel_flat[:BS]
    score_abnormal = ssel_flat[BS:]
    fs_flat = fs.reshape(NVID, NC)
    feat_select_normal = fs_flat[:BS].reshape(BS * NC)
    feat_select_abn = fs_flat[BS:].reshape(BS * NC)
    return (score_abnormal, score_normal, feat_select_abn, feat_select_normal,
            scores, s_loss.reshape(()), c_loss.reshape(()), u_loss.reshape(()))


# speedup vs baseline: 1.2422x; 1.0532x over previous
import jax
import jax.numpy as jnp
from jax import lax
from jax.experimental import pallas as pl
from jax.experimental.pallas import tpu as pltpu
from jax.experimental.pallas import tpu_sc as plsc

BS = 8
NC = 10
T = 32
F = 2048
H = 2048
MEM = 60
NVID = 2 * BS
RPV = NC * T
ROWS = NVID * RPV
K = T // 16 + 1
HF = F // 2
NSEL = NC * K


def _mlp_kernel(x_ref, wa1_ref, ba1_ref, wa2_ref, ba2_ref, w1_ref, b1_ref,
                w2_ref, b2_ref, w3_ref, b3_ref, out_ref, score_ref, fmag_ref):
    x = x_ref[...]
    h = jnp.maximum(
        jax.lax.dot(x, wa1_ref[...], preferred_element_type=jnp.float32)
        + ba1_ref[...], 0.0)
    out = x + jax.lax.dot(h, wa2_ref[...],
                          preferred_element_type=jnp.float32) + ba2_ref[...]
    out_ref[...] = out
    fmag_ref[...] = jnp.sqrt(jnp.sum(out * out, axis=1, keepdims=True))
    s = jnp.maximum(
        jax.lax.dot(out, w1_ref[...], preferred_element_type=jnp.float32)
        + b1_ref[...], 0.0)
    s = jnp.maximum(
        jax.lax.dot(s, w2_ref[...], preferred_element_type=jnp.float32)
        + b2_ref[...], 0.0)
    score_ref[...] = jax.nn.sigmoid(
        jax.lax.dot(s, w3_ref[...], preferred_element_type=jnp.float32)
        + b3_ref[...])


def _sel_kernel(fmag_ref, score_ref, mask_ref, scm_ref, ssel_ref, idx_ref):
    r_iota = jax.lax.broadcasted_iota(jnp.int32, (RPV, T), 0)
    t_iota = jax.lax.broadcasted_iota(jnp.int32, (RPV, T), 1)
    cm_mat = jnp.where(r_iota % T == t_iota, 1.0 / NC, 0.0)
    score_mean = jax.lax.dot(score_ref[...], cm_mat,
                             precision=jax.lax.Precision.HIGHEST,
                             preferred_element_type=jnp.float32)
    fmag_mean = jax.lax.dot(fmag_ref[...], cm_mat,
                            precision=jax.lax.Precision.HIGHEST,
                            preferred_element_type=jnp.float32)
    scm_ref[...] = score_mean
    mask_mean = jnp.mean(mask_ref[...], axis=1)
    drop = fmag_mean * mask_mean

    iota = jax.lax.broadcasted_iota(jnp.int32, (NVID, T), 1)
    v = drop
    wsel = jnp.zeros((NVID, T), jnp.float32)
    tks = []
    for _ in range(K):
        m = jnp.max(v, axis=1, keepdims=True)
        tk = jnp.min(jnp.where(v >= m, iota, T), axis=1, keepdims=True)
        hit = iota == tk
        wsel = wsel + jnp.where(hit, 1.0, 0.0)
        v = jnp.where(hit, -jnp.inf, v)
        tks.append(tk)

    ssel_ref[...] = jnp.sum(score_mean * wsel, axis=1, keepdims=True) / K

    col = iota
    c = jax.lax.shift_right_logical(col * 11, 5)
    kk = col - c * 3
    t = jnp.where(kk == 0, tks[0], jnp.where(kk == 1, tks[1], tks[2]))
    viota = jax.lax.broadcasted_iota(jnp.int32, (NVID, T), 0)
    row = viota * RPV + c * T + t
    idx_ref[...] = jnp.where(col < NSEL, row, 0)


def _sc_gather(feat_ref, idxrows_ref, sel_out, idx_vmem, idx_v, rows_v, sel_v,
               sem):
    wid = lax.axis_index("s") * 2 + lax.axis_index("c")
    v = wid // 2
    half = wid - 2 * v

    pltpu.sync_copy(idxrows_ref.at[v], idx_vmem)
    for g in range(2):
        r = idx_vmem[pl.ds(16 * g, 16)]
        idx_v[pl.ds(16 * g, 16)] = 2 * r + half
    pltpu.async_copy(feat_ref.at[idx_v], rows_v, sem).wait()

    def body(s, carry):
        off = s * 16
        for c in range(NC):
            acc = (rows_v[3 * c, pl.ds(off, 16)]
                   + rows_v[3 * c + 1, pl.ds(off, 16)]
                   + rows_v[3 * c + 2, pl.ds(off, 16)])
            sel_v[c, pl.ds(off, 16)] = acc * (1.0 / K)
        return carry

    lax.fori_loop(0, HF // 16, body, 0)

    for c in range(NC):
        pltpu.sync_copy(sel_v.at[c], sel_out.at[2 * (v * NC + c) + half])


def _loss_kernel(sel_ref, mem_ref, fs_ref, sacc_ref, cacc_ref, u_ref):
    sel = sel_ref[...]
    fs_ref[...] = jnp.sqrt(jnp.sum(sel * sel, axis=1, keepdims=True))

    mem = mem_ref[...]
    logits = jax.lax.dot_general(sel, mem, (((1,), (1,)), ((), ())),
                                 preferred_element_type=jnp.float32)
    att = jax.nn.softmax(logits, axis=-1)
    recon = jax.lax.dot(att, mem, preferred_element_type=jnp.float32)
    diff = recon - sel
    sacc_ref[...] = (jnp.sum(diff * diff) / (NVID * NC * F)).reshape(1, 1)
    cacc_ref[...] = (-jnp.sum(att * jnp.log(att + 1e-8))
                     / (NVID * NC)).reshape(1, 1)

    mn = mem / (jnp.sqrt(jnp.sum(mem * mem, axis=1, keepdims=True)) + 1e-8)
    g = jax.lax.dot_general(mn, mn, (((1,), (1,)), ((), ())),
                            preferred_element_type=jnp.float32)
    r = jax.lax.broadcasted_iota(jnp.int32, (MEM, MEM), 0)
    c = jax.lax.broadcasted_iota(jnp.int32, (MEM, MEM), 1)
    gd = g - jnp.where(r == c, 1.0, 0.0)
    u_ref[...] = (jnp.sum(gd * gd) / (MEM * MEM)).reshape(1, 1)


@jax.jit
def kernel(inputs, mask, Wa1, ba1, Wa2, ba2, W1, b1, W2, b2, W3, b3, Mem):
    x = inputs.reshape(ROWS, F)
    R = 512
    out, score_rows, fmag_rows = pl.pallas_call(
        _mlp_kernel,
        grid=(ROWS // R,),
        in_specs=[
            pl.BlockSpec((R, F), lambda i: (i, 0)),
            pl.BlockSpec((F, H), lambda i: (0, 0)),
            pl.BlockSpec((1, H), lambda i: (0, 0)),
            pl.BlockSpec((H, F), lambda i: (0, 0)),
            pl.BlockSpec((1, F), lambda i: (0, 0)),
            pl.BlockSpec((F, 512), lambda i: (0, 0)),
            pl.BlockSpec((1, 512), lambda i: (0, 0)),
            pl.BlockSpec((512, 128), lambda i: (0, 0)),
            pl.BlockSpec((1, 128), lambda i: (0, 0)),
            pl.BlockSpec((128, 1), lambda i: (0, 0)),
            pl.BlockSpec((1, 1), lambda i: (0, 0)),
        ],
        out_specs=[
            pl.BlockSpec((R, F), lambda i: (i, 0)),
            pl.BlockSpec((R, 1), lambda i: (i, 0)),
            pl.BlockSpec((R, 1), lambda i: (i, 0)),
        ],
        out_shape=[
            jax.ShapeDtypeStruct((ROWS, F), jnp.float32),
            jax.ShapeDtypeStruct((ROWS, 1), jnp.float32),
            jax.ShapeDtypeStruct((ROWS, 1), jnp.float32),
        ],
    )(x, Wa1, ba1.reshape(1, H), Wa2, ba2.reshape(1, F), W1,
      b1.reshape(1, 512), W2, b2.reshape(1, 128), W3, b3.reshape(1, 1))

    scm, ssel, idxrows = pl.pallas_call(
        _sel_kernel,
        grid=(1,),
        in_specs=[
            pl.BlockSpec((NVID, RPV), lambda i: (0, 0)),
            pl.BlockSpec((NVID, RPV), lambda i: (0, 0)),
            pl.BlockSpec((NVID, NC, T), lambda i: (0, 0, 0)),
        ],
        out_specs=[
            pl.BlockSpec((NVID, T), lambda i: (0, 0)),
            pl.BlockSpec((NVID, 1), lambda i: (0, 0)),
            pl.BlockSpec((NVID, T), lambda i: (0, 0)),
        ],
        out_shape=[
            jax.ShapeDtypeStruct((NVID, T), jnp.float32),
            jax.ShapeDtypeStruct((NVID, 1), jnp.float32),
            jax.ShapeDtypeStruct((NVID, T), jnp.int32),
        ],
    )(fmag_rows.reshape(NVID, RPV), score_rows.reshape(NVID, RPV),
      mask.reshape(NVID, NC, T))

    sc_fn = pl.kernel(
        _sc_gather,
        mesh=plsc.VectorSubcoreMesh(core_axis_name="c", subcore_axis_name="s"),
        out_type=jax.ShapeDtypeStruct((2 * NVID * NC, HF), jnp.float32),
        scratch_types=[
            pltpu.VMEM((T,), jnp.int32),
            pltpu.VMEM((T,), jnp.int32),
            pltpu.VMEM((T, HF), jnp.float32),
            pltpu.VMEM((NC, HF), jnp.float32),
            pltpu.SemaphoreType.DMA,
        ],
    )
    sel2 = sc_fn(out.reshape(2 * ROWS, HF), idxrows)
    sel = sel2.reshape(NVID * NC, F)

    fs, s_loss, c_loss, u_loss = pl.pallas_call(
        _loss_kernel,
        grid=(1,),
        in_specs=[
            pl.BlockSpec((NVID * NC, F), lambda i: (0, 0)),
            pl.BlockSpec((MEM, F), lambda i: (0, 0)),
        ],
        out_specs=[
            pl.BlockSpec((NVID * NC, 1), lambda i: (0, 0)),
            pl.BlockSpec((1, 1), lambda i: (0, 0)),
            pl.BlockSpec((1, 1), lambda i: (0, 0)),
            pl.BlockSpec((1, 1), lambda i: (0, 0)),
        ],
        out_shape=[
            jax.ShapeDtypeStruct((NVID * NC, 1), jnp.float32),
            jax.ShapeDtypeStruct((1, 1), jnp.float32),
            jax.ShapeDtypeStruct((1, 1), jnp.float32),
            jax.ShapeDtypeStruct((1, 1), jnp.float32),
        ],
    )(sel, Mem)

    scores = scm.reshape(NVID, T, 1)
    score_normal = ssel[:BS]
    score_abnormal = ssel[BS:]
    fs_flat = fs.reshape(NVID, NC)
    feat_select_normal = fs_flat[:BS].reshape(BS * NC)
    feat_select_abn = fs_flat[BS:].reshape(BS * NC)
    return (score_abnormal, score_normal, feat_select_abn, feat_select_normal,
            scores, s_loss.reshape(()), c_loss.reshape(()), u_loss.reshape(()))


# speedup vs baseline: 1.2583x; 1.0129x over previous
import jax
import jax.numpy as jnp
from jax import lax
from jax.experimental import pallas as pl
from jax.experimental.pallas import tpu as pltpu
from jax.experimental.pallas import tpu_sc as plsc

BS = 8
NC = 10
T = 32
F = 2048
H = 2048
MEM = 60
NVID = 2 * BS
RPV = NC * T
ROWS = NVID * RPV
K = T // 16 + 1
HF = F // 2
NEG = -3.0e38


def _mlp_kernel(x_ref, wa1_ref, ba1_ref, wa2_ref, ba2_ref, w1_ref, b1_ref,
                w2_ref, b2_ref, w3_ref, b3_ref, out_ref, score_ref, fmag_ref):
    x = x_ref[...]
    h = jnp.maximum(
        jax.lax.dot(x, wa1_ref[...], preferred_element_type=jnp.float32)
        + ba1_ref[...], 0.0)
    out = x + jax.lax.dot(h, wa2_ref[...],
                          preferred_element_type=jnp.float32) + ba2_ref[...]
    out_ref[...] = out
    fmag_ref[...] = jnp.sqrt(jnp.sum(out * out, axis=1, keepdims=True))
    s = jnp.maximum(
        jax.lax.dot(out, w1_ref[...], preferred_element_type=jnp.float32)
        + b1_ref[...], 0.0)
    s = jnp.maximum(
        jax.lax.dot(s, w2_ref[...], preferred_element_type=jnp.float32)
        + b2_ref[...], 0.0)
    score_ref[...] = jax.nn.sigmoid(
        jax.lax.dot(s, w3_ref[...], preferred_element_type=jnp.float32)
        + b3_ref[...])


def _sc_select(feat_ref, fmag_ref, score_ref, mask_ref,
               sel_out, scm_out, wsel_out,
               fmag_v, mask_v, score_v, vec32_v, idx_v, rows_v, sel_v, sem):
    wid = lax.axis_index("s") * 2 + lax.axis_index("c")
    v = wid // 2
    half = wid - 2 * v

    pltpu.sync_copy(fmag_ref.at[pl.ds(v * RPV, RPV)], fmag_v)
    pltpu.sync_copy(mask_ref.at[pl.ds(v * RPV, RPV)], mask_v)
    pltpu.sync_copy(score_ref.at[pl.ds(v * RPV, RPV)], score_v)

    tlo = lax.iota(jnp.int32, 16)
    zero = jnp.zeros((16,), jnp.float32)
    one = jnp.full((16,), 1.0)

    def crop_mean(ref, off):
        acc = zero
        for c in range(NC):
            acc = acc + ref[pl.ds(c * T + off, 16)]
        return acc * (1.0 / NC)

    fm0 = crop_mean(fmag_v, 0)
    fm1 = crop_mean(fmag_v, 16)
    mk0 = crop_mean(mask_v, 0)
    mk1 = crop_mean(mask_v, 16)
    a = fm0 * mk0
    b = fm1 * mk1

    red_v = vec32_v

    def rot(x, k):
        red_v[pl.ds(0, 16)] = x
        red_v[pl.ds(16, 16)] = x
        return red_v[pl.ds(k, 16)]

    def max_splat(x):
        for k in (8, 4, 2, 1):
            x = jnp.maximum(x, rot(x, k))
        return x

    def min_splat(x):
        for k in (8, 4, 2, 1):
            x = jnp.minimum(x, rot(x, k))
        return x

    tlo_f = tlo.astype(jnp.float32)
    ts = []
    for _ in range(K):
        ma = max_splat(a)
        mb = max_splat(b)
        pa_f = jnp.where(ma >= mb, one, zero)
        ca = min_splat(jnp.where(a >= ma, tlo_f, 99.0))
        cb = min_splat(jnp.where(b >= mb, tlo_f, 99.0))
        t_f = pa_f * ca + (one - pa_f) * (cb + 16.0)
        onehot_a = jnp.where(tlo_f == ca, one, zero) * pa_f
        onehot_b = jnp.where(tlo_f == cb, one, zero) * (one - pa_f)
        a = a + onehot_a * NEG
        b = b + onehot_b * NEG
        ts.append(t_f)

    @pl.when(half == 0)
    def _():
        sm0 = crop_mean(score_v, 0)
        sm1 = crop_mean(score_v, 16)
        vec32_v[pl.ds(0, 16)] = sm0
        vec32_v[pl.ds(16, 16)] = sm1
        pltpu.sync_copy(vec32_v, scm_out.at[v])

    @pl.when(half == 1)
    def _():
        w0 = zero
        w1 = zero
        for t in ts:
            w0 = w0 + jnp.where(tlo_f == t, 1.0, 0.0)
            w1 = w1 + jnp.where(tlo_f + 16.0 == t, 1.0, 0.0)
        vec32_v[pl.ds(0, 16)] = w0
        vec32_v[pl.ds(16, 16)] = w1
        pltpu.sync_copy(vec32_v, wsel_out.at[v])

    ts_i = [t.astype(jnp.int32) for t in ts]
    for g in range(2):
        j = tlo + 16 * g
        c = lax.shift_right_logical(j * 11, 5)
        kk = j - c * 3
        t = jnp.where(kk == 0, ts_i[0], jnp.where(kk == 1, ts_i[1], ts_i[2]))
        row = 2 * (v * RPV + c * T + t) + half
        row = jnp.where(j < NC * K, row, half)
        idx_v[pl.ds(16 * g, 16)] = row
    pltpu.async_copy(feat_ref.at[idx_v], rows_v, sem).wait()

    def body(s, carry):
        off = s * 16
        for c in range(NC):
            acc = (rows_v[3 * c, pl.ds(off, 16)]
                   + rows_v[3 * c + 1, pl.ds(off, 16)]
                   + rows_v[3 * c + 2, pl.ds(off, 16)])
            sel_v[c, pl.ds(off, 16)] = acc * (1.0 / K)
        return carry

    lax.fori_loop(0, HF // 16, body, 0)

    for c in range(NC):
        pltpu.sync_copy(sel_v.at[c], sel_out.at[2 * (v * NC + c) + half])


def _loss_kernel(sel_ref, scm_ref, wsel_ref, mem_ref,
                 ssel_ref, fs_ref, sacc_ref, cacc_ref, u_ref):
    sel = sel_ref[...]
    fs_ref[...] = jnp.sqrt(jnp.sum(sel * sel, axis=1, keepdims=True))
    scm = scm_ref[...]
    w = wsel_ref[...]
    ssel_ref[...] = jnp.sum(scm * w, axis=1, keepdims=True) / K

    mem = mem_ref[...]
    logits = jax.lax.dot_general(sel, mem, (((1,), (1,)), ((), ())),
                                 preferred_element_type=jnp.float32)
    att = jax.nn.softmax(logits, axis=-1)
    recon = jax.lax.dot(att, mem, preferred_element_type=jnp.float32)
    diff = recon - sel
    sacc_ref[...] = (jnp.sum(diff * diff) / (NVID * NC * F)).reshape(1, 1)
    cacc_ref[...] = (-jnp.sum(att * jnp.log(att + 1e-8))
                     / (NVID * NC)).reshape(1, 1)

    mn = mem / (jnp.sqrt(jnp.sum(mem * mem, axis=1, keepdims=True)) + 1e-8)
    g = jax.lax.dot_general(mn, mn, (((1,), (1,)), ((), ())),
                            preferred_element_type=jnp.float32)
    r = jax.lax.broadcasted_iota(jnp.int32, (MEM, MEM), 0)
    c = jax.lax.broadcasted_iota(jnp.int32, (MEM, MEM), 1)
    gd = g - jnp.where(r == c, 1.0, 0.0)
    u_ref[...] = (jnp.sum(gd * gd) / (MEM * MEM)).reshape(1, 1)


@jax.jit
def kernel(inputs, mask, Wa1, ba1, Wa2, ba2, W1, b1, W2, b2, W3, b3, Mem):
    x = inputs.reshape(ROWS, F)
    R = 512
    out, score_rows, fmag_rows = pl.pallas_call(
        _mlp_kernel,
        grid=(ROWS // R,),
        in_specs=[
            pl.BlockSpec((R, F), lambda i: (i, 0)),
            pl.BlockSpec((F, H), lambda i: (0, 0)),
            pl.BlockSpec((1, H), lambda i: (0, 0)),
            pl.BlockSpec((H, F), lambda i: (0, 0)),
            pl.BlockSpec((1, F), lambda i: (0, 0)),
            pl.BlockSpec((F, 512), lambda i: (0, 0)),
            pl.BlockSpec((1, 512), lambda i: (0, 0)),
            pl.BlockSpec((512, 128), lambda i: (0, 0)),
            pl.BlockSpec((1, 128), lambda i: (0, 0)),
            pl.BlockSpec((128, 1), lambda i: (0, 0)),
            pl.BlockSpec((1, 1), lambda i: (0, 0)),
        ],
        out_specs=[
            pl.BlockSpec((R, F), lambda i: (i, 0)),
            pl.BlockSpec((R, 1), lambda i: (i, 0)),
            pl.BlockSpec((R, 1), lambda i: (i, 0)),
        ],
        out_shape=[
            jax.ShapeDtypeStruct((ROWS, F), jnp.float32),
            jax.ShapeDtypeStruct((ROWS, 1), jnp.float32),
            jax.ShapeDtypeStruct((ROWS, 1), jnp.float32),
        ],
    )(x, Wa1, ba1.reshape(1, H), Wa2, ba2.reshape(1, F), W1,
      b1.reshape(1, 512), W2, b2.reshape(1, 128), W3, b3.reshape(1, 1))

    sc_fn = pl.kernel(
        _sc_select,
        mesh=plsc.VectorSubcoreMesh(core_axis_name="c", subcore_axis_name="s"),
        out_type=[
            jax.ShapeDtypeStruct((2 * NVID * NC, HF), jnp.float32),
            jax.ShapeDtypeStruct((NVID, T), jnp.float32),
            jax.ShapeDtypeStruct((NVID, T), jnp.float32),
        ],
        scratch_types=[
            pltpu.VMEM((RPV,), jnp.float32),
            pltpu.VMEM((RPV,), jnp.float32),
            pltpu.VMEM((RPV,), jnp.float32),
            pltpu.VMEM((T,), jnp.float32),
            pltpu.VMEM((2 * 16,), jnp.int32),
            pltpu.VMEM((2 * 16, HF), jnp.float32),
            pltpu.VMEM((NC, HF), jnp.float32),
            pltpu.SemaphoreType.DMA,
        ],
    )
    sel2, scm, wsel = sc_fn(out.reshape(2 * ROWS, HF),
                            fmag_rows.reshape(ROWS),
                            score_rows.reshape(ROWS),
                            mask.reshape(ROWS))
    sel = sel2.reshape(NVID * NC, F)

    ssel, fs, s_loss, c_loss, u_loss = pl.pallas_call(
        _loss_kernel,
        grid=(1,),
        in_specs=[
            pl.BlockSpec((NVID * NC, F), lambda i: (0, 0)),
            pl.BlockSpec((NVID, T), lambda i: (0, 0)),
            pl.BlockSpec((NVID, T), lambda i: (0, 0)),
            pl.BlockSpec((MEM, F), lambda i: (0, 0)),
        ],
        out_specs=[
            pl.BlockSpec((NVID, 1), lambda i: (0, 0)),
            pl.BlockSpec((NVID * NC, 1), lambda i: (0, 0)),
            pl.BlockSpec((1, 1), lambda i: (0, 0)),
            pl.BlockSpec((1, 1), lambda i: (0, 0)),
            pl.BlockSpec((1, 1), lambda i: (0, 0)),
        ],
        out_shape=[
            jax.ShapeDtypeStruct((NVID, 1), jnp.float32),
            jax.ShapeDtypeStruct((NVID * NC, 1), jnp.float32),
            jax.ShapeDtypeStruct((1, 1), jnp.float32),
            jax.ShapeDtypeStruct((1, 1), jnp.float32),
            jax.ShapeDtypeStruct((1, 1), jnp.float32),
        ],
    )(sel, scm, wsel, Mem)

    scores = scm.reshape(NVID, T, 1)
    score_normal = ssel[:BS]
    score_abnormal = ssel[BS:]
    fs_flat = fs.reshape(NVID, NC)
    feat_select_normal = fs_flat[:BS].reshape(BS * NC)
    feat_select_abn = fs_flat[BS:].reshape(BS * NC)
    return (score_abnormal, score_normal, feat_select_abn, feat_select_normal,
            scores, s_loss.reshape(()), c_loss.reshape(()), u_loss.reshape(()))


# speedup vs baseline: 1.2630x; 1.0038x over previous
import jax
import jax.numpy as jnp
from jax import lax
from jax.experimental import pallas as pl
from jax.experimental.pallas import tpu as pltpu
from jax.experimental.pallas import tpu_sc as plsc

BS = 8
NC = 10
T = 32
F = 2048
H = 2048
MEM = 60
NVID = 2 * BS
RPV = NC * T
ROWS = NVID * RPV
K = T // 16 + 1
HF = F // 2
NEG = -3.0e38


def _mlp_kernel(x_ref, wa1_ref, ba1_ref, wa2_ref, ba2_ref, w1_ref, b1_ref,
                w2_ref, b2_ref, w3_ref, b3_ref, out_ref, score_ref, fmag_ref):
    x = x_ref[...]
    h = jnp.maximum(
        jax.lax.dot(x, wa1_ref[...], preferred_element_type=jnp.float32)
        + ba1_ref[...], 0.0)
    out = x + jax.lax.dot(h, wa2_ref[...],
                          preferred_element_type=jnp.float32) + ba2_ref[...]
    out_ref[...] = out
    fmag_ref[...] = jnp.sqrt(jnp.sum(out * out, axis=1, keepdims=True))
    s = jnp.maximum(
        jax.lax.dot(out, w1_ref[...], preferred_element_type=jnp.float32)
        + b1_ref[...], 0.0)
    s = jnp.maximum(
        jax.lax.dot(s, w2_ref[...], preferred_element_type=jnp.float32)
        + b2_ref[...], 0.0)
    score_ref[...] = jax.nn.sigmoid(
        jax.lax.dot(s, w3_ref[...], preferred_element_type=jnp.float32)
        + b3_ref[...])


def _sc_select(feat_ref, fmag_ref, score_ref, mask_ref,
               sel_out, scm_out, wsel_out,
               fmag_v, mask_v, score_v, vec32_v, idx_v, rows_v, sel_v, sem):
    wid = lax.axis_index("s") * 2 + lax.axis_index("c")
    v = wid // 2
    half = wid - 2 * v

    cp1 = pltpu.async_copy(fmag_ref.at[pl.ds(v * RPV, RPV)], fmag_v, sem)
    cp2 = pltpu.async_copy(mask_ref.at[pl.ds(v * RPV, RPV)], mask_v, sem)
    cp3 = pltpu.async_copy(score_ref.at[pl.ds(v * RPV, RPV)], score_v, sem)
    cp1.wait()
    cp2.wait()
    cp3.wait()

    tlo = lax.iota(jnp.int32, 16)
    zero = jnp.zeros((16,), jnp.float32)
    one = jnp.full((16,), 1.0)

    def crop_mean(ref, off):
        acc = zero
        for c in range(NC):
            acc = acc + ref[pl.ds(c * T + off, 16)]
        return acc * (1.0 / NC)

    fm0 = crop_mean(fmag_v, 0)
    fm1 = crop_mean(fmag_v, 16)
    mk0 = crop_mean(mask_v, 0)
    mk1 = crop_mean(mask_v, 16)
    a = fm0 * mk0
    b = fm1 * mk1

    red_v = vec32_v

    def rot(x, k):
        red_v[pl.ds(0, 16)] = x
        red_v[pl.ds(16, 16)] = x
        return red_v[pl.ds(k, 16)]

    def max_splat(x):
        for k in (8, 4, 2, 1):
            x = jnp.maximum(x, rot(x, k))
        return x

    def min_splat(x):
        for k in (8, 4, 2, 1):
            x = jnp.minimum(x, rot(x, k))
        return x

    tlo_f = tlo.astype(jnp.float32)
    ts = []
    for _ in range(K):
        ma = max_splat(a)
        mb = max_splat(b)
        pa_f = jnp.where(ma >= mb, one, zero)
        ca = min_splat(jnp.where(a >= ma, tlo_f, 99.0))
        cb = min_splat(jnp.where(b >= mb, tlo_f, 99.0))
        t_f = pa_f * ca + (one - pa_f) * (cb + 16.0)
        onehot_a = jnp.where(tlo_f == ca, one, zero) * pa_f
        onehot_b = jnp.where(tlo_f == cb, one, zero) * (one - pa_f)
        a = a + onehot_a * NEG
        b = b + onehot_b * NEG
        ts.append(t_f)

    @pl.when(half == 0)
    def _():
        sm0 = crop_mean(score_v, 0)
        sm1 = crop_mean(score_v, 16)
        vec32_v[pl.ds(0, 16)] = sm0
        vec32_v[pl.ds(16, 16)] = sm1
        pltpu.sync_copy(vec32_v, scm_out.at[v])

    @pl.when(half == 1)
    def _():
        w0 = zero
        w1 = zero
        for t in ts:
            w0 = w0 + jnp.where(tlo_f == t, 1.0, 0.0)
            w1 = w1 + jnp.where(tlo_f + 16.0 == t, 1.0, 0.0)
        vec32_v[pl.ds(0, 16)] = w0
        vec32_v[pl.ds(16, 16)] = w1
        pltpu.sync_copy(vec32_v, wsel_out.at[v])

    ts_i = [t.astype(jnp.int32) for t in ts]
    for g in range(2):
        j = tlo + 16 * g
        c = lax.shift_right_logical(j * 11, 5)
        kk = j - c * 3
        t = jnp.where(kk == 0, ts_i[0], jnp.where(kk == 1, ts_i[1], ts_i[2]))
        row = 2 * (v * RPV + c * T + t) + half
        row = jnp.where(j < NC * K, row, half)
        idx_v[pl.ds(16 * g, 16)] = row
    pltpu.async_copy(feat_ref.at[idx_v], rows_v, sem).wait()

    def body(s, carry):
        off = s * 16
        for c in range(NC):
            acc = (rows_v[3 * c, pl.ds(off, 16)]
                   + rows_v[3 * c + 1, pl.ds(off, 16)]
                   + rows_v[3 * c + 2, pl.ds(off, 16)])
            sel_v[c, pl.ds(off, 16)] = acc * (1.0 / K)
        return carry

    lax.fori_loop(0, HF // 16, body, 0)

    pltpu.sync_copy(sel_v, sel_out.at[pl.ds((half * NVID + v) * 16, 16)])


def _loss_kernel(sel_lo_ref, sel_hi_ref, scm_ref, wsel_ref, mem_ref,
                 ssel_ref, fs_ref, sacc_ref, cacc_ref, u_ref):
    sel = jnp.concatenate([sel_lo_ref[...], sel_hi_ref[...]],
                          axis=1)
    fs_ref[...] = jnp.sqrt(jnp.sum(sel * sel, axis=1, keepdims=True))
    scm = scm_ref[...]
    w = wsel_ref[...]
    ssel_ref[...] = jnp.sum(scm * w, axis=1, keepdims=True) / K

    mem = mem_ref[...]
    logits = jax.lax.dot_general(sel, mem, (((1,), (1,)), ((), ())),
                                 preferred_element_type=jnp.float32)
    att = jax.nn.softmax(logits, axis=-1)
    recon = jax.lax.dot(att, mem, preferred_element_type=jnp.float32)
    diff = recon - sel
    sacc_ref[...] = (jnp.sum(diff * diff) / (NVID * NC * F)).reshape(1, 1)
    cacc_ref[...] = (-jnp.sum(att * jnp.log(att + 1e-8))
                     / (NVID * NC)).reshape(1, 1)

    mn = mem / (jnp.sqrt(jnp.sum(mem * mem, axis=1, keepdims=True)) + 1e-8)
    g = jax.lax.dot_general(mn, mn, (((1,), (1,)), ((), ())),
                            preferred_element_type=jnp.float32)
    r = jax.lax.broadcasted_iota(jnp.int32, (MEM, MEM), 0)
    c = jax.lax.broadcasted_iota(jnp.int32, (MEM, MEM), 1)
    gd = g - jnp.where(r == c, 1.0, 0.0)
    u_ref[...] = (jnp.sum(gd * gd) / (MEM * MEM)).reshape(1, 1)


@jax.jit
def kernel(inputs, mask, Wa1, ba1, Wa2, ba2, W1, b1, W2, b2, W3, b3, Mem):
    x = inputs.reshape(ROWS, F)
    R = 512
    out, score_rows, fmag_rows = pl.pallas_call(
        _mlp_kernel,
        grid=(ROWS // R,),
        in_specs=[
            pl.BlockSpec((R, F), lambda i: (i, 0)),
            pl.BlockSpec((F, H), lambda i: (0, 0)),
            pl.BlockSpec((1, H), lambda i: (0, 0)),
            pl.BlockSpec((H, F), lambda i: (0, 0)),
            pl.BlockSpec((1, F), lambda i: (0, 0)),
            pl.BlockSpec((F, 512), lambda i: (0, 0)),
            pl.BlockSpec((1, 512), lambda i: (0, 0)),
            pl.BlockSpec((512, 128), lambda i: (0, 0)),
            pl.BlockSpec((1, 128), lambda i: (0, 0)),
            pl.BlockSpec((128, 1), lambda i: (0, 0)),
            pl.BlockSpec((1, 1), lambda i: (0, 0)),
        ],
        out_specs=[
            pl.BlockSpec((R, F), lambda i: (i, 0)),
            pl.BlockSpec((R, 1), lambda i: (i, 0)),
            pl.BlockSpec((R, 1), lambda i: (i, 0)),
        ],
        out_shape=[
            jax.ShapeDtypeStruct((ROWS, F), jnp.float32),
            jax.ShapeDtypeStruct((ROWS, 1), jnp.float32),
            jax.ShapeDtypeStruct((ROWS, 1), jnp.float32),
        ],
    )(x, Wa1, ba1.reshape(1, H), Wa2, ba2.reshape(1, F), W1,
      b1.reshape(1, 512), W2, b2.reshape(1, 128), W3, b3.reshape(1, 1))

    sc_fn = pl.kernel(
        _sc_select,
        mesh=plsc.VectorSubcoreMesh(core_axis_name="c", subcore_axis_name="s"),
        out_type=[
            jax.ShapeDtypeStruct((2 * NVID * 16, HF), jnp.float32),
            jax.ShapeDtypeStruct((NVID, T), jnp.float32),
            jax.ShapeDtypeStruct((NVID, T), jnp.float32),
        ],
        scratch_types=[
            pltpu.VMEM((RPV,), jnp.float32),
            pltpu.VMEM((RPV,), jnp.float32),
            pltpu.VMEM((RPV,), jnp.float32),
            pltpu.VMEM((T,), jnp.float32),
            pltpu.VMEM((2 * 16,), jnp.int32),
            pltpu.VMEM((2 * 16, HF), jnp.float32),
            pltpu.VMEM((16, HF), jnp.float32),
            pltpu.SemaphoreType.DMA,
        ],
    )
    sel2, scm, wsel = sc_fn(out.reshape(2 * ROWS, HF),
                            fmag_rows.reshape(ROWS),
                            score_rows.reshape(ROWS),
                            mask.reshape(ROWS))
    sel3 = sel2.reshape(2, NVID, 16, HF)[:, :, :NC]
    sel_lo = sel3[0].reshape(NVID * NC, HF)
    sel_hi = sel3[1].reshape(NVID * NC, HF)

    ssel, fs, s_loss, c_loss, u_loss = pl.pallas_call(
        _loss_kernel,
        grid=(1,),
        in_specs=[
            pl.BlockSpec((NVID * NC, HF), lambda i: (0, 0)),
            pl.BlockSpec((NVID * NC, HF), lambda i: (0, 0)),
            pl.BlockSpec((NVID, T), lambda i: (0, 0)),
            pl.BlockSpec((NVID, T), lambda i: (0, 0)),
            pl.BlockSpec((MEM, F), lambda i: (0, 0)),
        ],
        out_specs=[
            pl.BlockSpec((NVID, 1), lambda i: (0, 0)),
            pl.BlockSpec((NVID * NC, 1), lambda i: (0, 0)),
            pl.BlockSpec((1, 1), lambda i: (0, 0)),
            pl.BlockSpec((1, 1), lambda i: (0, 0)),
            pl.BlockSpec((1, 1), lambda i: (0, 0)),
        ],
        out_shape=[
            jax.ShapeDtypeStruct((NVID, 1), jnp.float32),
            jax.ShapeDtypeStruct((NVID * NC, 1), jnp.float32),
            jax.ShapeDtypeStruct((1, 1), jnp.float32),
            jax.ShapeDtypeStruct((1, 1), jnp.float32),
            jax.ShapeDtypeStruct((1, 1), jnp.float32),
        ],
    )(sel_lo, sel_hi, scm, wsel, Mem)

    scores = scm.reshape(NVID, T, 1)
    score_normal = ssel[:BS]
    score_abnormal = ssel[BS:]
    fs_flat = fs.reshape(NVID, NC)
    feat_select_normal = fs_flat[:BS].reshape(BS * NC)
    feat_select_abn = fs_flat[BS:].reshape(BS * NC)
    return (score_abnormal, score_normal, feat_select_abn, feat_select_normal,
            scores, s_loss.reshape(()), c_loss.reshape(()), u_loss.reshape(()))
